# baseline (speedup 1.0000x reference)
import functools

import jax
import jax.numpy as jnp
from jax import lax
from jax.experimental import pallas as pl
from jax.experimental.pallas import tpu as pltpu
from jax.experimental.pallas import tpu_sc as plsc

L = 16


def _sc_meshinfo():
    info = plsc.get_sparse_core_info()
    NC, NS = info.num_cores, info.num_subcores
    mesh = plsc.VectorSubcoreMesh(core_axis_name="c", subcore_axis_name="s")
    return NC, NS, mesh


def _sc_vneg(V, D, B, N):
    W = 2 * D
    NC, NS, mesh = _sc_meshinfo()
    NW = NC * NS
    BW = B // NW
    C = 16
    NCH = BW // C
    G = D // L
    NEG_PER_GATHER = 128 // N * N
    n_gathers = (C * N + NEG_PER_GATHER - 1) // NEG_PER_GATHER

    @functools.partial(
        pl.kernel,
        out_type=jax.ShapeDtypeStruct((B, W), jnp.float32),
        mesh=mesh,
        scratch_types=[
            pltpu.VMEM((C,), jnp.int32),
            pltpu.VMEM((C * N,), jnp.int32),
            pltpu.VMEM((C, W), jnp.float32),
            pltpu.VMEM((C * N, W), jnp.float32),
            pltpu.VMEM((C, W), jnp.float32),
            pltpu.SemaphoreType.DMA,
            pltpu.SemaphoreType.DMA,
        ],
        compiler_params=pltpu.CompilerParams(
            use_tc_tiling_on_sc=False, needs_layout_passes=False),
    )
    def vneg_fn(targets_hbm, negflat_hbm, emb_v_hbm, vn_out,
                idx_v, idx_n, v_buf, n_buf, o_buf, sem_v, sem_n):
        wid = lax.axis_index("s") * NC + lax.axis_index("c")
        base = wid * BW

        def chunk_body(c, _):
            b0 = base + c * C
            pltpu.sync_copy(targets_hbm.at[pl.ds(b0, C)], idx_v)
            pltpu.sync_copy(negflat_hbm.at[pl.ds(b0 * N, C * N)], idx_n)

            cp_v = pltpu.async_copy(emb_v_hbm.at[idx_v], v_buf, sem_v)
            cps = []
            off = 0
            for _g in range(n_gathers):
                sz = min(NEG_PER_GATHER, C * N - off)
                cps.append(pltpu.async_copy(
                    emb_v_hbm.at[idx_n.at[pl.ds(off, sz)]],
                    n_buf.at[pl.ds(off, sz)], sem_n))
                off += sz
            cp_v.wait()
            for cp in cps:
                cp.wait()

            for k in range(C):
                for g in range(G):
                    o_buf[k, pl.ds(g * L, L)] = v_buf[k, pl.ds(D + g * L, L)]
                    sg = n_buf[k * N, pl.ds(D + g * L, L)]
                    for n in range(1, N):
                        sg = sg + n_buf[k * N + n, pl.ds(D + g * L, L)]
                    o_buf[k, pl.ds(D + g * L, L)] = sg
            pltpu.sync_copy(o_buf, vn_out.at[pl.ds(b0, C)])
            return 0

        lax.fori_loop(0, NCH, chunk_body, 0)

    return vneg_fn


def _sc_udot(V, D, B):
    W = 2 * D
    NC, NS, mesh = _sc_meshinfo()
    NW = NC * NS
    BW = B // NW
    C = 16
    NCH = BW // C
    G = D // L

    @functools.partial(
        pl.kernel,
        out_type=(
            jax.ShapeDtypeStruct((B, L), jnp.float32),
            jax.ShapeDtypeStruct((B, L), jnp.float32),
        ),
        mesh=mesh,
        scratch_types=[
            pltpu.VMEM((C,), jnp.int32),
            pltpu.VMEM((C, W), jnp.float32),
            pltpu.VMEM((C, W), jnp.float32),
            pltpu.VMEM((BW, L), jnp.float32),
            pltpu.VMEM((BW, L), jnp.float32),
            pltpu.SemaphoreType.DMA,
        ],
        compiler_params=pltpu.CompilerParams(
            use_tc_tiling_on_sc=False, needs_layout_passes=False),
    )
    def udot_fn(inputs_hbm, emb_u_hbm, vn_hbm, pos_out, neg_out,
                idx_u, u_buf, vn_buf, pos_acc, neg_acc, sem_u):
        wid = lax.axis_index("s") * NC + lax.axis_index("c")
        base = wid * BW

        def chunk_body(c, _):
            b0 = base + c * C
            pltpu.sync_copy(inputs_hbm.at[pl.ds(b0, C)], idx_u)
            cp_u = pltpu.async_copy(emb_u_hbm.at[idx_u], u_buf, sem_u)
            pltpu.sync_copy(vn_hbm.at[pl.ds(b0, C)], vn_buf)
            cp_u.wait()

            for k in range(C):
                pos = jnp.zeros((L,), jnp.float32)
                neg = jnp.zeros((L,), jnp.float32)
                for g in range(G):
                    ug = u_buf[k, pl.ds(g * L, L)]
                    pos = pos + ug * vn_buf[k, pl.ds(g * L, L)]
                    neg = neg + ug * vn_buf[k, pl.ds(D + g * L, L)]
                row = c * C + k
                pos_acc[row, :] = pos
                neg_acc[row, :] = neg
            return 0

        lax.fori_loop(0, NCH, chunk_body, 0)
        pltpu.sync_copy(pos_acc, pos_out.at[pl.ds(base, BW)])
        pltpu.sync_copy(neg_acc, neg_out.at[pl.ds(base, BW)])

    return udot_fn


def _tc_transpose_body(tin_u_ref, tin_v_ref, out_ref):
    eye = jnp.eye(64, dtype=jnp.float32)
    out_ref[:, 0:64] = jax.lax.dot_general(
        tin_u_ref[...], eye, (((0,), (0,)), ((), ())),
        preferred_element_type=jnp.float32)
    out_ref[:, 64:128] = jax.lax.dot_general(
        tin_v_ref[...], eye, (((0,), (0,)), ((), ())),
        preferred_element_type=jnp.float32)


def _tc_relayout(tin_u, tin_v, V, D):
    CB = 4096
    grid = (V + CB - 1) // CB
    return pl.pallas_call(
        _tc_transpose_body,
        grid=(grid,),
        in_specs=[
            pl.BlockSpec((D, CB), lambda i: (0, i)),
            pl.BlockSpec((D, CB), lambda i: (0, i)),
        ],
        out_specs=pl.BlockSpec((CB, 2 * D), lambda i: (i, 0)),
        out_shape=jax.ShapeDtypeStruct((V, 2 * D), jnp.float32),
    )(tin_u, tin_v)


def _tc_loss_body(pos_ref, neg_ref, out_ref):
    pos = jnp.sum(pos_ref[...], axis=1)
    neg = jnp.sum(neg_ref[...], axis=1)
    pls = jax.nn.log_sigmoid(pos)
    nls = jax.nn.log_sigmoid(neg)
    out_ref[0, 0] = -(jnp.mean(pls) - jnp.mean(nls))


def kernel(inputs, targets, negatives, emb_u, emb_v):
    V, D = emb_u.shape
    B = inputs.shape[0]
    N = negatives.shape[1]

    inputs = inputs.astype(jnp.int32)
    targets = targets.astype(jnp.int32)
    negflat = negatives.astype(jnp.int32).reshape(-1)

    comb = _tc_relayout(emb_u.T, emb_v.T, V, D)
    vn = _sc_vneg(V, D, B, N)(targets, negflat, comb)
    pos_part, neg_part = _sc_udot(V, D, B)(inputs, comb, vn)

    loss = pl.pallas_call(
        _tc_loss_body,
        out_shape=jax.ShapeDtypeStruct((1, 1), jnp.float32),
        out_specs=pl.BlockSpec(memory_space=pltpu.SMEM),
    )(pos_part, neg_part)
    return loss[0, 0]


# speedup vs baseline: 1.0292x; 1.0292x over previous
import functools

import jax
import jax.numpy as jnp
from jax import lax
from jax.experimental import pallas as pl
from jax.experimental.pallas import tpu as pltpu
from jax.experimental.pallas import tpu_sc as plsc

L = 16


def _sc_meshinfo():
    info = plsc.get_sparse_core_info()
    NC, NS = info.num_cores, info.num_subcores
    mesh = plsc.VectorSubcoreMesh(core_axis_name="c", subcore_axis_name="s")
    return NC, NS, mesh


def _sc_vneg(V, D, B, N):
    W = 2 * D
    NC, NS, mesh = _sc_meshinfo()
    NW = NC * NS
    BW = B // NW
    C = 16
    NCH = BW // C
    G = D // L
    NEG_PER_GATHER = 128 // N * N
    n_gathers = (C * N + NEG_PER_GATHER - 1) // NEG_PER_GATHER

    @functools.partial(
        pl.kernel,
        out_type=jax.ShapeDtypeStruct((B, W), jnp.float32),
        mesh=mesh,
        scratch_types=[
            pltpu.VMEM((C,), jnp.int32),
            pltpu.VMEM((C * N,), jnp.int32),
            pltpu.VMEM((C, W), jnp.float32),
            pltpu.VMEM((C * N, W), jnp.float32),
            pltpu.VMEM((C, W), jnp.float32),
            pltpu.SemaphoreType.DMA,
            pltpu.SemaphoreType.DMA,
        ],
        compiler_params=pltpu.CompilerParams(use_tc_tiling_on_sc=False),
    )
    def vneg_fn(targets_hbm, negflat_hbm, emb_v_hbm, vn_out,
                idx_v, idx_n, v_buf, n_buf, o_buf, sem_v, sem_n):
        wid = lax.axis_index("s") * NC + lax.axis_index("c")
        base = wid * BW

        def chunk_body(c, _):
            b0 = base + c * C
            pltpu.sync_copy(targets_hbm.at[pl.ds(b0, C)], idx_v)
            pltpu.sync_copy(negflat_hbm.at[pl.ds(b0 * N, C * N)], idx_n)

            cp_v = pltpu.async_copy(emb_v_hbm.at[idx_v], v_buf, sem_v)
            cps = []
            off = 0
            for _g in range(n_gathers):
                sz = min(NEG_PER_GATHER, C * N - off)
                cps.append(pltpu.async_copy(
                    emb_v_hbm.at[idx_n.at[pl.ds(off, sz)]],
                    n_buf.at[pl.ds(off, sz)], sem_n))
                off += sz
            cp_v.wait()
            for cp in cps:
                cp.wait()

            for k in range(C):
                for g in range(G):
                    o_buf[k, pl.ds(g * L, L)] = v_buf[k, pl.ds(g * L, L)]
                    sg = n_buf[k * N, pl.ds(g * L, L)]
                    for n in range(1, N):
                        sg = sg + n_buf[k * N + n, pl.ds(g * L, L)]
                    o_buf[k, pl.ds(D + g * L, L)] = sg
            pltpu.sync_copy(o_buf, vn_out.at[pl.ds(b0, C)])
            return 0

        lax.fori_loop(0, NCH, chunk_body, 0)

    return vneg_fn


def _sc_udot(V, D, B):
    W = 2 * D
    NC, NS, mesh = _sc_meshinfo()
    NW = NC * NS
    BW = B // NW
    C = 16
    NCH = BW // C
    G = D // L

    @functools.partial(
        pl.kernel,
        out_type=(
            jax.ShapeDtypeStruct((B, L), jnp.float32),
            jax.ShapeDtypeStruct((B, L), jnp.float32),
        ),
        mesh=mesh,
        scratch_types=[
            pltpu.VMEM((C,), jnp.int32),
            pltpu.VMEM((C, W), jnp.float32),
            pltpu.VMEM((C, W), jnp.float32),
            pltpu.VMEM((BW, L), jnp.float32),
            pltpu.VMEM((BW, L), jnp.float32),
            pltpu.SemaphoreType.DMA,
        ],
        compiler_params=pltpu.CompilerParams(use_tc_tiling_on_sc=False),
    )
    def udot_fn(inputs_hbm, emb_u_hbm, vn_hbm, pos_out, neg_out,
                idx_u, u_buf, vn_buf, pos_acc, neg_acc, sem_u):
        wid = lax.axis_index("s") * NC + lax.axis_index("c")
        base = wid * BW

        def chunk_body(c, _):
            b0 = base + c * C
            pltpu.sync_copy(inputs_hbm.at[pl.ds(b0, C)], idx_u)
            cp_u = pltpu.async_copy(emb_u_hbm.at[idx_u], u_buf, sem_u)
            pltpu.sync_copy(vn_hbm.at[pl.ds(b0, C)], vn_buf)
            cp_u.wait()

            for k in range(C):
                pos = jnp.zeros((L,), jnp.float32)
                neg = jnp.zeros((L,), jnp.float32)
                for g in range(G):
                    ug = u_buf[k, pl.ds(g * L, L)]
                    pos = pos + ug * vn_buf[k, pl.ds(g * L, L)]
                    neg = neg + ug * vn_buf[k, pl.ds(D + g * L, L)]
                row = c * C + k
                pos_acc[row, :] = pos
                neg_acc[row, :] = neg
            return 0

        lax.fori_loop(0, NCH, chunk_body, 0)
        pltpu.sync_copy(pos_acc, pos_out.at[pl.ds(base, BW)])
        pltpu.sync_copy(neg_acc, neg_out.at[pl.ds(base, BW)])

    return udot_fn


def _tc_transpose_body(tin_ref, out_ref):
    t = jnp.transpose(tin_ref[...], (1, 0))
    out_ref[:, 0:t.shape[1]] = t
    out_ref[:, t.shape[1]:] = jnp.zeros_like(t)


def _tc_relayout(tin, V, D):
    CB = 4096
    grid = (V + CB - 1) // CB
    return pl.pallas_call(
        _tc_transpose_body,
        grid=(grid,),
        in_specs=[pl.BlockSpec((D, CB), lambda i: (0, i))],
        out_specs=pl.BlockSpec((CB, 2 * D), lambda i: (i, 0)),
        out_shape=jax.ShapeDtypeStruct((V, 2 * D), jnp.float32),
    )(tin)


def _tc_loss_body(pos_ref, neg_ref, out_ref):
    pos = jnp.sum(pos_ref[...], axis=1)
    neg = jnp.sum(neg_ref[...], axis=1)
    pls = jax.nn.log_sigmoid(pos)
    nls = jax.nn.log_sigmoid(neg)
    out_ref[0, 0] = -(jnp.mean(pls) - jnp.mean(nls))


def kernel(inputs, targets, negatives, emb_u, emb_v):
    V, D = emb_u.shape
    B = inputs.shape[0]
    N = negatives.shape[1]

    inputs = inputs.astype(jnp.int32)
    targets = targets.astype(jnp.int32)
    negflat = negatives.astype(jnp.int32).reshape(-1)

    emb_v_p = _tc_relayout(emb_v.T, V, D)
    vn = _sc_vneg(V, D, B, N)(targets, negflat, emb_v_p)
    emb_u_p = _tc_relayout(emb_u.T, V, D)
    pos_part, neg_part = _sc_udot(V, D, B)(inputs, emb_u_p, vn)

    loss = pl.pallas_call(
        _tc_loss_body,
        out_shape=jax.ShapeDtypeStruct((1, 1), jnp.float32),
        out_specs=pl.BlockSpec(memory_space=pltpu.SMEM),
    )(pos_part, neg_part)
    return loss[0, 0]
